# baseline (speedup 1.0000x reference)
import functools

import jax
import jax.numpy as jnp
import numpy as np
from jax import lax
from jax.experimental import pallas as pl
from jax.experimental.pallas import tpu as pltpu
from jax.experimental.pallas import tpu_sc as plsc

C, H, W = 192, 512, 512
NC, NS = 2, 16
NW = NC * NS
CW = C // NW

_PERM = np.asarray(jax.random.permutation(jax.random.key(42), C))

_mesh = plsc.VectorSubcoreMesh(core_axis_name="c", subcore_axis_name="s")


@functools.partial(
    pl.kernel,
    out_type=jax.ShapeDtypeStruct((C, H, W), jnp.float32),
    mesh=_mesh,
    scratch_types=[pltpu.SemaphoreType.DMA],
)
def _permute_channels(img_hbm, out_hbm, sem):
    wid = lax.axis_index("s") * NC + lax.axis_index("c")

    for w in range(NW):
        def issue(w=w):
            for j in range(CW):
                dst_c = w * CW + j
                src_c = int(_PERM[dst_c])
                pltpu.make_async_copy(
                    img_hbm.at[src_c], out_hbm.at[dst_c], sem
                ).start()
        pl.when(wid == w)(issue)

    for j in range(CW):
        pltpu.make_async_copy(img_hbm.at[0], out_hbm.at[0], sem).wait()


def kernel(img):
    return _permute_channels(img)


# speedup vs baseline: 9.8618x; 9.8618x over previous
import functools

import jax
import jax.numpy as jnp
from jax import lax
from jax.experimental import pallas as pl
from jax.experimental.pallas import tpu as pltpu
from jax.experimental.pallas import tpu_sc as plsc

C, H, W = 192, 512, 512
SPLIT = 8
SR = C * SPLIT
SW = H * W // SPLIT
NC, NS = 2, 16
NW = NC * NS
NCHUNK = SR // NW
NBUF = 3
NGRP = NCHUNK // NBUF

_mesh = plsc.VectorSubcoreMesh(core_axis_name="c", subcore_axis_name="s")


@functools.partial(
    pl.kernel,
    out_type=jax.ShapeDtypeStruct((SR, SW), jnp.float32),
    mesh=_mesh,
    scratch_types=[
        pltpu.VMEM((NCHUNK, 1), jnp.int32),
        [pltpu.VMEM((1, SW), jnp.float32)] * NBUF,
        [pltpu.SemaphoreType.DMA] * NBUF,
        [pltpu.SemaphoreType.DMA] * NBUF,
    ],
)
def _permute_rows(img_hbm, idx_hbm, out_hbm, idx_v, bufs, isems, osems):
    wid = lax.axis_index("s") * NC + lax.axis_index("c")
    pltpu.sync_copy(idx_hbm.at[wid], idx_v)
    base = wid * NCHUNK

    def start_in(k, b):
        pltpu.make_async_copy(img_hbm.at[idx_v.at[k]], bufs[b], isems[b]).start()

    def wait_in(b):
        pltpu.make_async_copy(img_hbm.at[idx_v.at[0]], bufs[b], isems[b]).wait()

    def start_out(k, b):
        dst = out_hbm.at[pl.ds(base + k, 1), :]
        pltpu.make_async_copy(bufs[b], dst, osems[b]).start()

    def wait_out(b):
        dst = out_hbm.at[pl.ds(base, 1), :]
        pltpu.make_async_copy(bufs[b], dst, osems[b]).wait()

    start_in(0, 0)
    start_in(1, 1)

    def body(g, carry):
        for b in range(NBUF):
            k = g * NBUF + b
            b2 = (b + 2) % NBUF
            wait_in(b)
            start_out(k, b)
            pl.when(k >= 1)(lambda: wait_out(b2))
            pl.when(k + 2 < NCHUNK)(lambda: start_in(k + 2, b2))
        return carry

    lax.fori_loop(0, NGRP, body, 0)
    wait_out((NCHUNK - 1) % NBUF)


def kernel(img):
    perm = jax.random.permutation(jax.random.key(42), C)
    sr_idx = (perm[:, None] * SPLIT + jnp.arange(SPLIT)[None, :]).astype(jnp.int32)
    idx = sr_idx.reshape(NW, NCHUNK, 1)
    out2 = _permute_rows(img.reshape(SR, SW), idx)
    return out2.reshape(C, H, W)


# speedup vs baseline: 10.2216x; 1.0365x over previous
import functools

import jax
import jax.numpy as jnp
from jax import lax
from jax.experimental import pallas as pl
from jax.experimental.pallas import tpu as pltpu
from jax.experimental.pallas import tpu_sc as plsc

C, H, W = 192, 512, 512
SPLIT = 128
R = C * SPLIT
RWID = H * W // SPLIT
NC, NS = 2, 16
NW = NC * NS
RW = R // NW
CHUNK = 16
NCHUNK = RW // CHUNK
NBUF = 3
NGRP = NCHUNK // NBUF

assert R % NW == 0 and RW % CHUNK == 0 and NCHUNK % NBUF == 0

_mesh = plsc.VectorSubcoreMesh(core_axis_name="c", subcore_axis_name="s")


@functools.partial(
    pl.kernel,
    out_type=jax.ShapeDtypeStruct((R, RWID), jnp.float32),
    mesh=_mesh,
    scratch_types=[
        pltpu.VMEM((NCHUNK, CHUNK), jnp.int32),
        [pltpu.VMEM((CHUNK, RWID), jnp.float32)] * NBUF,
        [pltpu.SemaphoreType.DMA] * NBUF,
        [pltpu.SemaphoreType.DMA] * NBUF,
    ],
)
def _permute_rows(img_hbm, idx_hbm, out_hbm, idx_v, bufs, isems, osems):
    wid = lax.axis_index("s") * NC + lax.axis_index("c")
    pltpu.sync_copy(idx_hbm.at[wid], idx_v)
    base = wid * RW

    def start_in(k, b):
        pltpu.make_async_copy(img_hbm.at[idx_v.at[k]], bufs[b], isems[b]).start()

    def wait_in(b):
        pltpu.make_async_copy(img_hbm.at[idx_v.at[0]], bufs[b], isems[b]).wait()

    def start_out(k, b):
        dst = out_hbm.at[pl.ds(base + k * CHUNK, CHUNK), :]
        pltpu.make_async_copy(bufs[b], dst, osems[b]).start()

    def wait_out(b):
        dst = out_hbm.at[pl.ds(base, CHUNK), :]
        pltpu.make_async_copy(bufs[b], dst, osems[b]).wait()

    start_in(0, 0)
    start_in(1, 1)

    def body(g, carry):
        for b in range(NBUF):
            k = g * NBUF + b
            b2 = (b + 2) % NBUF
            wait_in(b)
            start_out(k, b)
            pl.when(k >= 1)(lambda: wait_out(b2))
            pl.when(k + 2 < NCHUNK)(lambda: start_in(k + 2, b2))
        return carry

    lax.fori_loop(0, NGRP, body, 0)
    wait_out((NCHUNK - 1) % NBUF)


def kernel(img):
    perm = jax.random.permutation(jax.random.key(42), C)
    row_idx = (perm[:, None] * SPLIT + jnp.arange(SPLIT)[None, :]).astype(jnp.int32)
    idx = row_idx.reshape(NW, NCHUNK, CHUNK)
    out2 = _permute_rows(img.reshape(R, RWID), idx)
    return out2.reshape(C, H, W)


# speedup vs baseline: 37.0290x; 3.6226x over previous
import functools

import jax
import jax.numpy as jnp
from jax import lax
from jax.experimental import pallas as pl
from jax.experimental.pallas import tpu as pltpu
from jax.experimental.pallas import tpu_sc as plsc

C, H, W = 192, 512, 512
G = 64
BPC = H // G
NB = C * BPC
NC, NS = 2, 16
NW = NC * NS
CHUNK = 1
NCHUNK = NB // NW
NBUF = 3
NGRP = NCHUNK // NBUF

assert NB % NW == 0 and NCHUNK % NBUF == 0

_mesh = plsc.VectorSubcoreMesh(core_axis_name="c", subcore_axis_name="s")


@functools.partial(
    pl.kernel,
    out_type=jax.ShapeDtypeStruct((NB, G, W), jnp.float32),
    mesh=_mesh,
    scratch_types=[
        pltpu.VMEM((NCHUNK, CHUNK), jnp.int32),
        [pltpu.VMEM((CHUNK, G, W), jnp.float32)] * NBUF,
        [pltpu.SemaphoreType.DMA] * NBUF,
        [pltpu.SemaphoreType.DMA] * NBUF,
    ],
)
def _permute_rows(img_hbm, idx_hbm, out_hbm, idx_v, bufs, isems, osems):
    wid = lax.axis_index("s") * NC + lax.axis_index("c")
    pltpu.sync_copy(idx_hbm.at[wid], idx_v)
    base = wid * NCHUNK

    def start_in(k, b):
        pltpu.make_async_copy(img_hbm.at[idx_v.at[k]], bufs[b], isems[b]).start()

    def wait_in(b):
        pltpu.make_async_copy(img_hbm.at[idx_v.at[0]], bufs[b], isems[b]).wait()

    def start_out(k, b):
        dst = out_hbm.at[pl.ds(base + k * CHUNK, CHUNK), :, :]
        pltpu.make_async_copy(bufs[b], dst, osems[b]).start()

    def wait_out(b):
        dst = out_hbm.at[pl.ds(base, CHUNK), :, :]
        pltpu.make_async_copy(bufs[b], dst, osems[b]).wait()

    start_in(0, 0)
    start_in(1, 1)

    def body(g, carry):
        for b in range(NBUF):
            k = g * NBUF + b
            b2 = (b + 2) % NBUF
            wait_in(b)
            start_out(k, b)
            pl.when(k >= 1)(lambda: wait_out(b2))
            pl.when(k + 2 < NCHUNK)(lambda: start_in(k + 2, b2))
        return carry

    lax.fori_loop(0, NGRP, body, 0)
    wait_out((NCHUNK - 1) % NBUF)


def kernel(img):
    perm = jax.random.permutation(jax.random.key(42), C)
    blk_idx = (perm[:, None] * BPC + jnp.arange(BPC)[None, :]).astype(jnp.int32)
    idx = blk_idx.reshape(NW, NCHUNK, CHUNK)
    out2 = _permute_rows(img.reshape(NB, G, W), idx)
    return out2.reshape(C, H, W)
